# baseline (speedup 1.0000x reference)
import jax
import jax.numpy as jnp
from jax import lax
from jax.experimental import pallas as pl

_NUM_E = 1024
_DIM = 64
_BETA = 0.25


def _vq_body(x_ref, cb2_ref, cb_ref, e2_ref, q_ref, idx_ref, loss_ref):
    b = pl.program_id(0)
    CB2 = cb2_ref[...]
    CB = cb_ref[...]
    e2 = e2_ref[...]

    s = jnp.zeros((8, 128), jnp.float32)
    for i in range(x_ref.shape[0]):
        X = x_ref[i]
        x2 = jnp.sum(X * X, axis=0, keepdims=True)

        mm2 = lax.dot_general(CB2, X, (((1,), (0,)), ((), ())))
        HWn = X.shape[1]
        G, GN = 256, _NUM_E // 256
        giota = lax.broadcasted_iota(jnp.int32, (G, HWn), 0)

        def d_chunk(g):
            return (x2 + e2[g * G:(g + 1) * G]) - mm2[g * G:(g + 1) * G]

        parts = [jnp.min(d_chunk(g), axis=0, keepdims=True) for g in range(GN)]
        m = parts[0]
        for pm in parts[1:]:
            m = jnp.minimum(m, pm)

        iparts = []
        for g in range(GN):
            loc = jnp.min(jnp.where(d_chunk(g) == m, giota, _NUM_E),
                          axis=0, keepdims=True)
            iparts.append(jnp.minimum(loc + g * G, _NUM_E))
        idx = iparts[0]
        for ip in iparts[1:]:
            idx = jnp.minimum(idx, ip)
        idx_ref[i] = idx

        eidx = lax.broadcasted_iota(jnp.int32, (_NUM_E, HWn), 0)
        E = (eidx == idx).astype(jnp.float32)
        q = lax.dot_general(CB, E, (((0,), (0,)), ((), ())))
        q_ref[i] = q

        s = s + jnp.full((8, 128), jnp.sum(m), jnp.float32)

    @pl.when(b == 0)
    def _init():
        loss_ref[...] = s

    @pl.when(b != 0)
    def _acc():
        loss_ref[...] = loss_ref[...] + s


def kernel(inputs, codebook):
    B, C, H, W = inputs.shape
    HW = H * W
    xr = inputs.reshape(B, C, HW)
    e2 = jnp.sum(codebook ** 2, axis=1).reshape(_NUM_E, 1)
    cb2 = 2.0 * codebook

    PB = 2
    q, idx, loss_acc = pl.pallas_call(
        _vq_body,
        grid=(B // PB,),
        in_specs=[
            pl.BlockSpec((PB, C, HW), lambda b: (b, 0, 0)),
            pl.BlockSpec((_NUM_E, C), lambda b: (0, 0)),
            pl.BlockSpec((_NUM_E, C), lambda b: (0, 0)),
            pl.BlockSpec((_NUM_E, 1), lambda b: (0, 0)),
        ],
        out_specs=[
            pl.BlockSpec((PB, C, HW), lambda b: (b, 0, 0)),
            pl.BlockSpec((PB, 1, HW), lambda b: (b, 0, 0)),
            pl.BlockSpec((8, 128), lambda b: (0, 0)),
        ],
        out_shape=[
            jax.ShapeDtypeStruct((B, C, HW), jnp.float32),
            jax.ShapeDtypeStruct((B, 1, HW), jnp.int32),
            jax.ShapeDtypeStruct((8, 128), jnp.float32),
        ],
    )(xr, cb2, codebook, e2)

    quantized_out = q.reshape(B, C, H, W)
    encoding_indices = idx.reshape(B * HW)
    e_latent = loss_acc[0, 0] / (B * HW * C)
    vq_loss = e_latent + _BETA * e_latent
    return quantized_out, vq_loss, encoding_indices


# speedup vs baseline: 1.0030x; 1.0030x over previous
import jax
import jax.numpy as jnp
from jax import lax
from jax.experimental import pallas as pl

_NUM_E = 1024
_DIM = 64
_BETA = 0.25


def _vq_body(x_ref, cb2_ref, cb_ref, e2_ref, q_ref, idx_ref, loss_ref):
    b = pl.program_id(0)
    CB2 = cb2_ref[...]
    CB = cb_ref[...]
    e2 = e2_ref[...]

    s = jnp.zeros((8, 128), jnp.float32)
    for i in range(x_ref.shape[0]):
        X = x_ref[i]
        x2 = jnp.sum(X * X, axis=0, keepdims=True)

        mm2 = lax.dot_general(CB2, X, (((1,), (0,)), ((), ())))
        HWn = X.shape[1]
        G, GN = 128, _NUM_E // 128
        giota = lax.broadcasted_iota(jnp.int32, (G, HWn), 0)

        def d_chunk(g):
            return (x2 + e2[g * G:(g + 1) * G]) - mm2[g * G:(g + 1) * G]

        parts = [jnp.min(d_chunk(g), axis=0, keepdims=True) for g in range(GN)]
        m = parts[0]
        for pm in parts[1:]:
            m = jnp.minimum(m, pm)

        iparts = []
        for g in range(GN):
            loc = jnp.min(jnp.where(d_chunk(g) == m, giota, _NUM_E),
                          axis=0, keepdims=True)
            iparts.append(jnp.minimum(loc + g * G, _NUM_E))
        idx = iparts[0]
        for ip in iparts[1:]:
            idx = jnp.minimum(idx, ip)
        idx_ref[i] = idx

        eidx = lax.broadcasted_iota(jnp.int32, (_NUM_E, HWn), 0)
        E = (eidx == idx).astype(jnp.float32)
        q = lax.dot_general(CB, E, (((0,), (0,)), ((), ())))
        q_ref[i] = q

        s = s + jnp.full((8, 128), jnp.sum(m), jnp.float32)

    @pl.when(b == 0)
    def _init():
        loss_ref[...] = s

    @pl.when(b != 0)
    def _acc():
        loss_ref[...] = loss_ref[...] + s


def kernel(inputs, codebook):
    B, C, H, W = inputs.shape
    HW = H * W
    xr = inputs.reshape(B, C, HW)
    e2 = jnp.sum(codebook ** 2, axis=1).reshape(_NUM_E, 1)
    cb2 = 2.0 * codebook

    PB = 2
    q, idx, loss_acc = pl.pallas_call(
        _vq_body,
        grid=(B // PB,),
        in_specs=[
            pl.BlockSpec((PB, C, HW), lambda b: (b, 0, 0)),
            pl.BlockSpec((_NUM_E, C), lambda b: (0, 0)),
            pl.BlockSpec((_NUM_E, C), lambda b: (0, 0)),
            pl.BlockSpec((_NUM_E, 1), lambda b: (0, 0)),
        ],
        out_specs=[
            pl.BlockSpec((PB, C, HW), lambda b: (b, 0, 0)),
            pl.BlockSpec((PB, 1, HW), lambda b: (b, 0, 0)),
            pl.BlockSpec((8, 128), lambda b: (0, 0)),
        ],
        out_shape=[
            jax.ShapeDtypeStruct((B, C, HW), jnp.float32),
            jax.ShapeDtypeStruct((B, 1, HW), jnp.int32),
            jax.ShapeDtypeStruct((8, 128), jnp.float32),
        ],
    )(xr, cb2, codebook, e2)

    quantized_out = q.reshape(B, C, H, W)
    encoding_indices = idx.reshape(B * HW)
    e_latent = loss_acc[0, 0] / (B * HW * C)
    vq_loss = e_latent + _BETA * e_latent
    return quantized_out, vq_loss, encoding_indices
